# baseline (speedup 1.0000x reference)
import functools

import jax
import jax.numpy as jnp
from jax import lax
from jax.experimental import pallas as pl
from jax.experimental.pallas import tpu as pltpu
from jax.experimental.pallas import tpu_sc as plsc

VOCAB = 1000
D = 1000
DA = 896
DB = D - DA
B, T = 4096, 50
TP = 56
NC, NS = 2, 16
NW = NC * NS
B_PER_W = B // NW


def _sc_gather(table3, idx3):
    mesh = plsc.VectorSubcoreMesh(core_axis_name="c", subcore_axis_name="s")

    @functools.partial(
        pl.kernel,
        mesh=mesh,
        out_type=jax.ShapeDtypeStruct((B, T, D), jnp.float32),
        scratch_types=[
            pltpu.VMEM((B_PER_W, TP), jnp.int32),
            pltpu.VMEM((TP, 8, 128), jnp.float32),
            pltpu.VMEM((T, DA), jnp.float32),
            pltpu.VMEM((T, DB), jnp.float32),
            pltpu.SemaphoreType.DMA,
            pltpu.SemaphoreType.DMA,
            pltpu.SemaphoreType.DMA,
        ],
    )
    def k(t3_hbm, idx_hbm, out_hbm, idx_v, g3, bufa, bufc, gs, wa, wb):
        sid = lax.axis_index("s")
        wid = sid * NC + lax.axis_index("c")
        base = wid * B_PER_W
        pltpu.sync_copy(idx_hbm.at[wid], idx_v)

        def start_g(j):
            pltpu.async_copy(t3_hbm.at[idx_v.at[j]], g3, gs)

        def wait_g(j):
            pltpu.make_async_copy(t3_hbm.at[idx_v.at[j]], g3, gs).wait()

        def unpack():
            def row_copy(t, carry):
                for c in range(7):
                    for m in range(8):
                        bufa[t, pl.ds(c * 128 + m * 16, 16)] = (
                            g3[t, c, pl.ds(m * 16, 16)])
                for m in range(6):
                    bufc[t, pl.ds(m * 16, 16)] = g3[t, 7, pl.ds(m * 16, 16)]
                bufc[t, pl.ds(88, 16)] = g3[t, 7, pl.ds(88, 16)]
                return carry

            lax.fori_loop(0, T, row_copy, 0)

        def start_w(j):
            bi = base + j
            pltpu.async_copy(bufa, out_hbm.at[bi, :, pl.ds(0, DA)], wa)
            pltpu.async_copy(bufc, out_hbm.at[bi, :, pl.ds(DA, DB)], wb)

        def wait_w(j):
            bi = base + j
            pltpu.make_async_copy(
                bufa, out_hbm.at[bi, :, pl.ds(0, DA)], wa).wait()
            pltpu.make_async_copy(
                bufc, out_hbm.at[bi, :, pl.ds(DA, DB)], wb).wait()

        start_g(0)

        def body(j, carry):
            wait_g(j)
            wait_w(j - 1)
            unpack()
            start_g(j + 1)
            start_w(j)
            return carry

        wait_g(0)
        unpack()
        start_g(1)
        start_w(0)
        lax.fori_loop(1, B_PER_W - 1, body, 0)
        j_last = B_PER_W - 1
        wait_g(j_last)
        wait_w(j_last - 1)
        unpack()
        start_w(j_last)
        wait_w(j_last)

    return k(table3, idx3)


def kernel(idx, table):
    idxp = jnp.pad(idx.astype(jnp.int32), ((0, 0), (0, TP - T)))
    idx3 = idxp.reshape(NW, B_PER_W, TP)
    table3 = jnp.pad(table, ((0, 0), (0, 24))).reshape(VOCAB, 8, 128)
    return _sc_gather(table3, idx3)


# speedup vs baseline: 1.3999x; 1.3999x over previous
import functools

import jax
import jax.numpy as jnp
from jax import lax
from jax.experimental import pallas as pl
from jax.experimental.pallas import tpu as pltpu
from jax.experimental.pallas import tpu_sc as plsc

VOCAB = 1000
D = 1000
B, T = 4096, 50
N_ROWS = B * T
NC, NS = 2, 16
NW = NC * NS
ROWS_PER_W = N_ROWS // NW
CHUNK = 32
N_CHUNKS = ROWS_PER_W // CHUNK
N_HALVES = 2
HALF = N_CHUNKS // N_HALVES


def _sc_gather(table, idx4):
    mesh = plsc.VectorSubcoreMesh(core_axis_name="c", subcore_axis_name="s")

    @functools.partial(
        pl.kernel,
        mesh=mesh,
        out_type=jax.ShapeDtypeStruct((N_ROWS, D), jnp.float32),
        compiler_params=pltpu.CompilerParams(use_tc_tiling_on_sc=False),
        scratch_types=[
            pltpu.VMEM((HALF, CHUNK), jnp.int32),
            pltpu.VMEM((CHUNK, D), jnp.float32),
            pltpu.VMEM((CHUNK, D), jnp.float32),
            pltpu.VMEM_SHARED((VOCAB, D), jnp.float32),
            pltpu.SemaphoreType.DMA,
            pltpu.SemaphoreType.DMA,
            pltpu.SemaphoreType.DMA,
            pltpu.SemaphoreType.DMA,
        ],
    )
    def k(table_hbm, idx_hbm, out_hbm, idx_v, buf0, buf1, table_sp,
          g0, g1, w0, w1):
        sid = lax.axis_index("s")
        wid = sid * NC + lax.axis_index("c")
        base = wid * ROWS_PER_W

        @pl.when(sid == 0)
        def _():
            pltpu.sync_copy(table_hbm, table_sp)

        plsc.subcore_barrier()

        bufs = (buf0, buf1)
        gsems = (g0, g1)
        wsems = (w0, w1)

        def half_pass(h):
            pltpu.sync_copy(idx_hbm.at[wid, h], idx_v)
            off0 = base + h * HALF * CHUNK

            def start_gather(j, b):
                pltpu.async_copy(table_sp.at[idx_v.at[j]], bufs[b], gsems[b])

            def wait_gather(j, b):
                pltpu.make_async_copy(
                    table_sp.at[idx_v.at[j]], bufs[b], gsems[b]).wait()

            def start_write(j, b):
                pltpu.async_copy(
                    bufs[b], out_hbm.at[pl.ds(off0 + j * CHUNK, CHUNK)],
                    wsems[b])

            def wait_write(j, b):
                pltpu.make_async_copy(
                    bufs[b], out_hbm.at[pl.ds(off0 + j * CHUNK, CHUNK)],
                    wsems[b]).wait()

            def step(j, b):
                wait_gather(j, b)
                start_write(j, b)
                wait_write(j - 1, 1 - b)
                start_gather(j + 1, 1 - b)

            start_gather(0, 0)
            wait_gather(0, 0)
            start_write(0, 0)
            start_gather(1, 1)

            def body(jj, carry):
                step(2 * jj + 1, 1)
                step(2 * jj + 2, 0)
                return carry

            lax.fori_loop(0, (HALF - 4) // 2, body, 0)

            step(HALF - 3, 1)
            step(HALF - 2, 0)
            j_last = HALF - 1
            wait_gather(j_last, 1)
            start_write(j_last, 1)
            wait_write(j_last - 1, 0)
            wait_write(j_last, 1)

        for h in range(N_HALVES):
            half_pass(h)

    return k(table, idx4)


def kernel(idx, table):
    flat = idx.reshape(-1).astype(jnp.int32)
    idx4 = flat.reshape(NW, N_HALVES, HALF, CHUNK)
    out = _sc_gather(table, idx4)
    return out.reshape(B, T, D)
